# baseline (speedup 1.0000x reference)
import functools

import jax
import jax.numpy as jnp
from jax import lax
from jax.experimental import pallas as pl
from jax.experimental.pallas import tpu as pltpu
from jax.experimental.pallas import tpu_sc as plsc


def _bcast16(vec, lane):
    idx = jnp.full((16, 1), lane, dtype=jnp.int32)
    dnums = lax.GatherDimensionNumbers(
        offset_dims=(), collapsed_slice_dims=(0,), start_index_map=(0,))
    return lax.gather(vec, idx, dnums, (1,),
                      mode=lax.GatherScatterMode.PROMISE_IN_BOUNDS)


_NC = 2
_NS = 16
_CW = 80


def _sc_aggregate(npad, d, y2, col3, row3, ev3, b2):
    cps = col3.shape[1]
    rpt = npad // _NS
    dh = d // _NC

    mesh = plsc.VectorSubcoreMesh(core_axis_name="c", subcore_axis_name="s")
    nbuf = 5
    assert cps % nbuf == 0
    assert rpt % 16 == 0

    @functools.partial(
        pl.kernel,
        out_type=jax.ShapeDtypeStruct((_NC, npad, dh), jnp.float32),
        mesh=mesh,
        scratch_types=[
            pltpu.VMEM((cps, _CW), jnp.int32),
            pltpu.VMEM((cps, _CW), jnp.int32),
            pltpu.VMEM((cps, _CW), jnp.float32),
            pltpu.VMEM((nbuf, _CW, dh), jnp.float32),
            pltpu.VMEM((dh,), jnp.float32),
            pltpu.VMEM((16, dh), jnp.float32),
            pltpu.VMEM_SHARED((npad, dh), jnp.float32),
            pltpu.SemaphoreType.DMA((nbuf,)),
            pltpu.SemaphoreType.DMA((nbuf,)),
        ],
        compiler_params=pltpu.CompilerParams(use_tc_tiling_on_sc=False),
    )
    def body(y_hbm, col_hbm, row_hbm, ev_hbm, b_hbm, out_hbm,
             col_v, row_v, ev_v, rows_v, brow_v, btile_v, agg_sh,
             gsems, ssems):
        c = lax.axis_index("c")
        s = lax.axis_index("s")

        pltpu.sync_copy(col_hbm.at[s], col_v)
        pltpu.sync_copy(row_hbm.at[s], row_v)
        pltpu.sync_copy(ev_hbm.at[s], ev_v)
        pltpu.sync_copy(b_hbm.at[c], brow_v)
        for q in range(dh // 16):
            seg = brow_v[pl.ds(q * 16, 16)]
            for r in range(16):
                btile_v[r, pl.ds(q * 16, 16)] = seg

        def seed_body(i, carry):
            pltpu.sync_copy(btile_v, agg_sh.at[pl.ds(s * rpt + i * 16, 16)])
            return carry

        lax.fori_loop(0, rpt // 16, seed_body, 0)
        plsc.subcore_barrier()

        def g_start(j, b):
            pltpu.async_copy(y_hbm.at[c].at[col_v.at[j]], rows_v.at[b],
                             gsems.at[b])

        def g_wait(b):
            pltpu.make_async_copy(y_hbm.at[c].at[pl.ds(0, _CW)],
                                  rows_v.at[b], gsems.at[b]).wait()

        def s_start(j, b):
            pltpu.async_copy(rows_v.at[b], agg_sh.at[row_v.at[j]],
                             ssems.at[b], add=True)

        def s_wait(b):
            pltpu.make_async_copy(y_hbm.at[c].at[pl.ds(0, _CW)],
                                  rows_v.at[b], ssems.at[b]).wait()

        for b in range(nbuf):
            g_start(b, b)

        def group_body(p, carry):
            for b in range(nbuf):
                j = p * nbuf + b
                g_wait(b)
                rbuf = rows_v.at[b]

                jb = j - 2
                pb = (b - 2) % nbuf

                @pl.when(jb >= 0)
                def _():
                    s_wait(pb)

                    @pl.when(jb + nbuf < cps)
                    def _():
                        g_start(jb + nbuf, pb)

                for blk in range(_CW // 16):
                    ev16 = ev_v[j, pl.ds(blk * 16, 16)]
                    for lane in range(16):
                        evb = _bcast16(ev16, lane)
                        ei = blk * 16 + lane
                        for r in range(dh // 16):
                            sl = pl.ds(r * 16, 16)
                            rbuf[ei, sl] = rbuf[ei, sl] * evb

                s_start(j, b)

            return carry

        lax.fori_loop(0, cps // nbuf, group_body, 0)
        s_wait(nbuf - 2)
        s_wait(nbuf - 1)
        plsc.subcore_barrier()

        pltpu.sync_copy(agg_sh.at[pl.ds(s * rpt, rpt)],
                        out_hbm.at[c, pl.ds(s * rpt, rpt)])

    return body(y2, col3, row3, ev3, b2)


def _proj_body(x_ref, w_ref, o_ref):
    y = lax.dot_general(x_ref[...], w_ref[...], (((1,), (1,)), ((), ())),
                        preferred_element_type=jnp.float32)
    dh = y.shape[1] // 2
    o_ref[0] = y[:, :dh]
    o_ref[1] = y[:, dh:]


def _tc_project(n, d, x, w):
    rb = 1000
    dh = d // _NC
    return pl.pallas_call(
        _proj_body,
        grid=(n // rb,),
        in_specs=[
            pl.BlockSpec((rb, d), lambda i: (i, 0)),
            pl.BlockSpec((d, d), lambda i: (0, 0)),
        ],
        out_specs=pl.BlockSpec((_NC, rb, dh), lambda i: (0, i, 0)),
        out_shape=jax.ShapeDtypeStruct((_NC, n, dh), jnp.float32),
    )(x, w)


def kernel(x, edge_index, edge_values, W, b):
    n, d = x.shape
    e = edge_values.shape[0]
    dh = d // _NC
    cps = e // (_NS * _CW)
    npad = ((n + _NS * 16 - 1) // (_NS * 16)) * (_NS * 16)
    row3 = edge_index[0].reshape(_NS, cps, _CW)
    col3 = edge_index[1].reshape(_NS, cps, _CW)
    ev3 = edge_values.reshape(_NS, cps, _CW)
    b2 = b.reshape(_NC, dh)
    y2 = _tc_project(n, d, x, W)
    halves = _sc_aggregate(npad, d, y2, col3, row3, ev3, b2)
    return jnp.concatenate([halves[0, :n], halves[1, :n]], axis=1)


# speedup vs baseline: 1.1058x; 1.1058x over previous
import functools

import jax
import jax.numpy as jnp
from jax import lax
from jax.experimental import pallas as pl
from jax.experimental.pallas import tpu as pltpu
from jax.experimental.pallas import tpu_sc as plsc


def _bcast16(vec, lane):
    idx = jnp.full((16, 1), lane, dtype=jnp.int32)
    dnums = lax.GatherDimensionNumbers(
        offset_dims=(), collapsed_slice_dims=(0,), start_index_map=(0,))
    return lax.gather(vec, idx, dnums, (1,),
                      mode=lax.GatherScatterMode.PROMISE_IN_BOUNDS)


_NC = 2
_NS = 16
_CW = 80


def _sc_aggregate(n, npad, d, y2, col3, row3, ev3, b2):
    cps = col3.shape[1]
    rpt = npad // _NS
    dh = d // _NC
    tail = n - (_NS - 1) * rpt
    assert 0 < tail <= rpt and tail % 8 == 0

    mesh = plsc.VectorSubcoreMesh(core_axis_name="c", subcore_axis_name="s")
    nbuf = 5
    assert cps % nbuf == 0
    assert rpt % 16 == 0

    @functools.partial(
        pl.kernel,
        out_type=jax.ShapeDtypeStruct((n, d), jnp.float32),
        mesh=mesh,
        scratch_types=[
            pltpu.VMEM((cps, _CW), jnp.int32),
            pltpu.VMEM((cps, _CW), jnp.int32),
            pltpu.VMEM((cps, _CW), jnp.float32),
            pltpu.VMEM((nbuf, _CW, dh), jnp.float32),
            pltpu.VMEM((dh,), jnp.float32),
            pltpu.VMEM((16, dh), jnp.float32),
            pltpu.VMEM_SHARED((npad, dh), jnp.float32),
            pltpu.SemaphoreType.DMA((nbuf,)),
            pltpu.SemaphoreType.DMA((nbuf,)),
        ],
        compiler_params=pltpu.CompilerParams(use_tc_tiling_on_sc=False),
    )
    def body(y_hbm, col_hbm, row_hbm, ev_hbm, b_hbm, out_hbm,
             col_v, row_v, ev_v, rows_v, brow_v, btile_v, agg_sh,
             gsems, ssems):
        c = lax.axis_index("c")
        s = lax.axis_index("s")

        pltpu.sync_copy(col_hbm.at[s], col_v)
        pltpu.sync_copy(row_hbm.at[s], row_v)
        pltpu.sync_copy(ev_hbm.at[s], ev_v)
        pltpu.sync_copy(b_hbm.at[c], brow_v)
        for q in range(dh // 16):
            seg = brow_v[pl.ds(q * 16, 16)]
            for r in range(16):
                btile_v[r, pl.ds(q * 16, 16)] = seg

        def seed_body(i, carry):
            pltpu.sync_copy(btile_v, agg_sh.at[pl.ds(s * rpt + i * 16, 16)])
            return carry

        lax.fori_loop(0, rpt // 16, seed_body, 0)
        plsc.subcore_barrier()

        def g_start(j, b):
            pltpu.async_copy(y_hbm.at[c].at[col_v.at[j]], rows_v.at[b],
                             gsems.at[b])

        def g_wait(b):
            pltpu.make_async_copy(y_hbm.at[c].at[pl.ds(0, _CW)],
                                  rows_v.at[b], gsems.at[b]).wait()

        def s_start(j, b):
            pltpu.async_copy(rows_v.at[b], agg_sh.at[row_v.at[j]],
                             ssems.at[b], add=True)

        def s_wait(b):
            pltpu.make_async_copy(y_hbm.at[c].at[pl.ds(0, _CW)],
                                  rows_v.at[b], ssems.at[b]).wait()

        for b in range(nbuf):
            g_start(b, b)

        def group_body(p, carry):
            for b in range(nbuf):
                j = p * nbuf + b
                g_wait(b)
                rbuf = rows_v.at[b]

                jb = j - 2
                pb = (b - 2) % nbuf

                @pl.when(jb >= 0)
                def _():
                    s_wait(pb)

                    @pl.when(jb + nbuf < cps)
                    def _():
                        g_start(jb + nbuf, pb)

                for blk in range(_CW // 16):
                    ev16 = ev_v[j, pl.ds(blk * 16, 16)]
                    for lane in range(16):
                        evb = _bcast16(ev16, lane)
                        ei = blk * 16 + lane
                        for r in range(dh // 16):
                            sl = pl.ds(r * 16, 16)
                            rbuf[ei, sl] = rbuf[ei, sl] * evb

                s_start(j, b)

            return carry

        lax.fori_loop(0, cps // nbuf, group_body, 0)
        s_wait(nbuf - 2)
        s_wait(nbuf - 1)
        plsc.subcore_barrier()

        @pl.when(s < _NS - 1)
        def _():
            pltpu.sync_copy(agg_sh.at[pl.ds(s * rpt, rpt)],
                            out_hbm.at[pl.ds(s * rpt, rpt), pl.ds(c * dh, dh)])

        @pl.when(s == _NS - 1)
        def _():
            pltpu.sync_copy(agg_sh.at[pl.ds(s * rpt, tail)],
                            out_hbm.at[pl.ds(s * rpt, tail), pl.ds(c * dh, dh)])

    return body(y2, col3, row3, ev3, b2)


def _proj_body(x_ref, w_ref, o_ref):
    y = lax.dot_general(x_ref[...], w_ref[...], (((1,), (1,)), ((), ())),
                        preferred_element_type=jnp.float32)
    dh = y.shape[1] // 2
    o_ref[0] = y[:, :dh]
    o_ref[1] = y[:, dh:]


def _tc_project(n, d, x, w):
    rb = 1000
    dh = d // _NC
    return pl.pallas_call(
        _proj_body,
        grid=(n // rb,),
        in_specs=[
            pl.BlockSpec((rb, d), lambda i: (i, 0)),
            pl.BlockSpec((d, d), lambda i: (0, 0)),
        ],
        out_specs=pl.BlockSpec((_NC, rb, dh), lambda i: (0, i, 0)),
        out_shape=jax.ShapeDtypeStruct((_NC, n, dh), jnp.float32),
    )(x, w)


def kernel(x, edge_index, edge_values, W, b):
    n, d = x.shape
    e = edge_values.shape[0]
    dh = d // _NC
    cps = e // (_NS * _CW)
    npad = ((n + _NS * 16 - 1) // (_NS * 16)) * (_NS * 16)
    row3 = edge_index[0].reshape(_NS, cps, _CW)
    col3 = edge_index[1].reshape(_NS, cps, _CW)
    ev3 = edge_values.reshape(_NS, cps, _CW)
    b2 = b.reshape(_NC, dh)
    y2 = _tc_project(n, d, x, W)
    return _sc_aggregate(n, npad, d, y2, col3, row3, ev3, b2)
